# Initial kernel scaffold; baseline (speedup 1.0000x reference)
import jax
import jax.numpy as jnp
from jax.experimental import pallas as pl


def kernel(input_ids, table, W1, b1, W2, b2):
    raise NotImplementedError("write your pallas kernel here")


# speedup vs baseline: 7.3087x; 7.3087x over previous
import functools

import jax
import jax.numpy as jnp
from jax import lax
from jax.experimental import pallas as pl
from jax.experimental.pallas import tpu as pltpu
from jax.experimental.pallas import tpu_sc as plsc

B = 16384
L = 200
E = 32
NCLS = 50
NCLS_PAD = 64
NW = 32
BPW = B // NW

_mesh = plsc.VectorSubcoreMesh(core_axis_name="c", subcore_axis_name="s")


@functools.partial(
    pl.kernel,
    out_type=jax.ShapeDtypeStruct((B, E), jnp.float32),
    mesh=_mesh,
    scratch_types=[
        pltpu.VMEM((L,), jnp.int32),
        pltpu.VMEM((L, E), jnp.float32),
        pltpu.VMEM((E,), jnp.float32),
        pltpu.SemaphoreType.DMA,
    ],
    compiler_params=pltpu.CompilerParams(use_tc_tiling_on_sc=False),
)
def _pool_sc(ids_hbm, table_hbm, out_hbm, idx_v, rows_v, acc_v, sem):
    wid = lax.axis_index("s") * 2 + lax.axis_index("c")
    base = wid * BPW

    def row_body(i, carry):
        r = base + i
        pltpu.sync_copy(ids_hbm.at[r], idx_v)
        cp1 = pltpu.async_copy(
            table_hbm.at[idx_v.at[pl.ds(0, 128)]], rows_v.at[pl.ds(0, 128)], sem)
        cp2 = pltpu.async_copy(
            table_hbm.at[idx_v.at[pl.ds(128, 72)]], rows_v.at[pl.ds(128, 72)], sem)
        cp1.wait()
        cp2.wait()

        def red(j, accs):
            a0, a1 = accs
            return a0 + rows_v[j, pl.ds(0, 16)], a1 + rows_v[j, pl.ds(16, 16)]

        z = jnp.zeros((16,), jnp.float32)
        a0, a1 = lax.fori_loop(0, L, red, (z, z))
        acc_v[pl.ds(0, 16)] = a0
        acc_v[pl.ds(16, 16)] = a1
        pltpu.sync_copy(acc_v, out_hbm.at[r])
        return carry

    lax.fori_loop(0, BPW, row_body, 0)


def _mlp_body(x_ref, w1_ref, b1_ref, w2_ref, b2_ref, o_ref):
    h = jnp.dot(x_ref[...], w1_ref[...], preferred_element_type=jnp.float32)
    h = jnp.maximum(h + b1_ref[...], 0.0)
    o_ref[...] = jnp.dot(h, w2_ref[...], preferred_element_type=jnp.float32) + b2_ref[...]


_BM = 2048

_mlp = pl.pallas_call(
    _mlp_body,
    grid=(B // _BM,),
    in_specs=[
        pl.BlockSpec((_BM, E), lambda i: (i, 0)),
        pl.BlockSpec((E, 128), lambda i: (0, 0)),
        pl.BlockSpec((1, 128), lambda i: (0, 0)),
        pl.BlockSpec((128, NCLS_PAD), lambda i: (0, 0)),
        pl.BlockSpec((1, NCLS_PAD), lambda i: (0, 0)),
    ],
    out_specs=pl.BlockSpec((_BM, NCLS_PAD), lambda i: (i, 0)),
    out_shape=jax.ShapeDtypeStruct((B, NCLS_PAD), jnp.float32),
)


def kernel(input_ids, table, W1, b1, W2, b2):
    pooled = _pool_sc(input_ids.astype(jnp.int32), table)
    w1s = W1.T.astype(jnp.float32) * (1.0 / L)
    b1r = b1.reshape(1, 128)
    w2p = jnp.pad(W2.T, ((0, 0), (0, NCLS_PAD - NCLS)))
    b2p = jnp.pad(b2, (0, NCLS_PAD - NCLS)).reshape(1, NCLS_PAD)
    out = _mlp(pooled, w1s, b1r, w2p, b2p)
    return out[:, :NCLS]


# speedup vs baseline: 12.9655x; 1.7740x over previous
import functools

import jax
import jax.numpy as jnp
from jax import lax
from jax.experimental import pallas as pl
from jax.experimental.pallas import tpu as pltpu
from jax.experimental.pallas import tpu_sc as plsc

B = 16384
L = 200
E = 32
NCLS = 50
NCLS_PAD = 64
NW = 32
BPW = B // NW

_mesh = plsc.VectorSubcoreMesh(core_axis_name="c", subcore_axis_name="s")

R = 8
NCHUNK = BPW // R
_SPLITS = ((0, 128), (128, 72))


@functools.partial(
    pl.kernel,
    out_type=jax.ShapeDtypeStruct((B, E), jnp.float32),
    mesh=_mesh,
    scratch_types=[
        pltpu.VMEM((R, L), jnp.int32),
        pltpu.VMEM((R, L), jnp.int32),
        pltpu.VMEM((R, L, E), jnp.float32),
        pltpu.VMEM((R, L, E), jnp.float32),
        pltpu.VMEM((BPW, E), jnp.float32),
        pltpu.SemaphoreType.DMA,
        pltpu.SemaphoreType.DMA,
        pltpu.SemaphoreType.DMA,
    ],
    compiler_params=pltpu.CompilerParams(use_tc_tiling_on_sc=False),
)
def _pool_sc(ids_hbm, table_hbm, out_hbm, ibuf0, ibuf1, rbuf0, rbuf1,
             obuf, sem_i, sem_g0, sem_g1):
    wid = lax.axis_index("s") * 2 + lax.axis_index("c")
    base = wid * BPW

    def fire_idx(c, ibuf):
        pltpu.make_async_copy(ids_hbm.at[pl.ds(base + c * R, R)], ibuf, sem_i).start()

    def wait_idx(ibuf):
        pltpu.make_async_copy(ids_hbm.at[pl.ds(base, R)], ibuf, sem_i).wait()

    def fire_gathers(ibuf, rbuf, sem):
        for r in range(R):
            for (o, w) in _SPLITS:
                pltpu.make_async_copy(
                    table_hbm.at[ibuf.at[r, pl.ds(o, w)]],
                    rbuf.at[r, pl.ds(o, w)], sem).start()

    def wait_gathers(ibuf, rbuf, sem):
        for r in range(R):
            for (o, w) in _SPLITS:
                pltpu.make_async_copy(
                    table_hbm.at[ibuf.at[r, pl.ds(o, w)]],
                    rbuf.at[r, pl.ds(o, w)], sem).wait()

    def reduce_chunk(c, rbuf):
        for r in range(R):
            def red(j, accs):
                a0, a1 = accs
                return a0 + rbuf[r, j, pl.ds(0, 16)], a1 + rbuf[r, j, pl.ds(16, 16)]
            z = jnp.zeros((16,), jnp.float32)
            a0, a1 = lax.fori_loop(0, L, red, (z, z))
            row = c * R + r
            obuf[row, pl.ds(0, 16)] = a0
            obuf[row, pl.ds(16, 16)] = a1

    fire_idx(0, ibuf0)

    def body(c2, carry):
        c = 2 * c2
        wait_idx(ibuf0)
        fire_gathers(ibuf0, rbuf0, sem_g0)

        @pl.when(c2 > 0)
        def _():
            wait_gathers(ibuf1, rbuf1, sem_g1)

        fire_idx(c + 1, ibuf1)

        @pl.when(c2 > 0)
        def _():
            reduce_chunk(c - 1, rbuf1)

        wait_idx(ibuf1)
        fire_gathers(ibuf1, rbuf1, sem_g1)
        wait_gathers(ibuf0, rbuf0, sem_g0)

        @pl.when(c2 < NCHUNK // 2 - 1)
        def _():
            fire_idx(c + 2, ibuf0)

        reduce_chunk(c, rbuf0)
        return carry

    lax.fori_loop(0, NCHUNK // 2, body, 0)
    wait_gathers(ibuf1, rbuf1, sem_g1)
    reduce_chunk(NCHUNK - 1, rbuf1)
    pltpu.sync_copy(obuf, out_hbm.at[pl.ds(base, BPW)])


def _mlp_body(x_ref, w1_ref, b1_ref, w2_ref, b2_ref, o_ref):
    h = jnp.dot(x_ref[...], w1_ref[...], preferred_element_type=jnp.float32)
    h = jnp.maximum(h + b1_ref[...], 0.0)
    o_ref[...] = jnp.dot(h, w2_ref[...], preferred_element_type=jnp.float32) + b2_ref[...]


_BM = 2048

_mlp = pl.pallas_call(
    _mlp_body,
    grid=(B // _BM,),
    in_specs=[
        pl.BlockSpec((_BM, E), lambda i: (i, 0)),
        pl.BlockSpec((E, 128), lambda i: (0, 0)),
        pl.BlockSpec((1, 128), lambda i: (0, 0)),
        pl.BlockSpec((128, NCLS_PAD), lambda i: (0, 0)),
        pl.BlockSpec((1, NCLS_PAD), lambda i: (0, 0)),
    ],
    out_specs=pl.BlockSpec((_BM, NCLS_PAD), lambda i: (i, 0)),
    out_shape=jax.ShapeDtypeStruct((B, NCLS_PAD), jnp.float32),
)


def kernel(input_ids, table, W1, b1, W2, b2):
    pooled = _pool_sc(input_ids.astype(jnp.int32), table)
    w1s = W1.T.astype(jnp.float32) * (1.0 / L)
    b1r = b1.reshape(1, 128)
    w2p = jnp.pad(W2.T, ((0, 0), (0, NCLS_PAD - NCLS)))
    b2p = jnp.pad(b2, (0, NCLS_PAD - NCLS)).reshape(1, NCLS_PAD)
    out = _mlp(pooled, w1s, b1r, w2p, b2p)
    return out[:, :NCLS]
